# baseline (speedup 1.0000x reference)
import functools

import jax
import jax.numpy as jnp
from jax import lax
from jax.experimental import pallas as pl
from jax.experimental.pallas import tpu as pltpu
from jax.experimental.pallas import tpu_sc as plsc

FN = jnp.float32

N = 10000
NP = 10240
NT = 16
RPT = NP // NT
E_RAW = 160000
EF = E_RAW + N
K = 80
NB = 136
NB2 = NB // 2
W = 4
NW = NB // W
NW2 = NW // 2
EP = NT * NB * K
R = 1024
GRID = NP // R
EPS = 1e-3
CLS = 40


def _prop_body(cps, hp, src_c, dst_w, zeros, out, src_v, dwin_a, dwin_b,
               rows_a, rows_b, sem_da, sem_db, sem_a, sem_b, acc):
    cid = lax.axis_index("c")
    tid = lax.axis_index("s")

    def pair(j, dwin, ta, tb, last):
        pltpu.async_copy(hp.at[src_v.at[pl.ds((j + 1) * K, K)]],
                         rows_b, sem_b)
        pltpu.make_async_copy(hp.at[src_v.at[pl.ds(0, K)]],
                              rows_a, sem_a).wait()
        pltpu.sync_copy(rows_a, acc.at[dwin.at[ta]], add=True)
        if last:
            @pl.when(j + 2 < NB)
            def _():
                pltpu.async_copy(hp.at[src_v.at[pl.ds((j + 2) * K, K)]],
                                 rows_a, sem_a)
        else:
            pltpu.async_copy(hp.at[src_v.at[pl.ds((j + 2) * K, K)]],
                             rows_a, sem_a)
        pltpu.make_async_copy(hp.at[src_v.at[pl.ds(0, K)]],
                              rows_b, sem_b).wait()
        pltpu.sync_copy(rows_b, acc.at[dwin.at[tb]], add=True)

    for lc in range(cps):
        chunk = cid * cps + lc
        pltpu.sync_copy(src_c.at[chunk * NT + tid], src_v)
        pltpu.async_copy(dst_w.at[tid * NW], dwin_a, sem_da)
        pltpu.async_copy(hp.at[src_v.at[pl.ds(0, K)]], rows_a, sem_a)
        pltpu.sync_copy(zeros.at[pl.ds(tid * RPT, RPT)],
                        acc.at[pl.ds(tid * RPT, RPT)])
        plsc.subcore_barrier()

        @pl.loop(0, NW2, unroll=False)
        def outer(w2):
            j0 = 2 * W * w2
            wa = 2 * w2
            pltpu.async_copy(dst_w.at[tid * NW + wa + 1], dwin_b, sem_db)
            pltpu.make_async_copy(dst_w.at[tid * NW], dwin_a, sem_da).wait()
            pair(j0, dwin_a, 0, 1, False)
            pair(j0 + 2, dwin_a, 2, 3, False)

            @pl.when(w2 < NW2 - 1)
            def _():
                pltpu.async_copy(dst_w.at[tid * NW + wa + 2], dwin_a, sem_da)

            pltpu.make_async_copy(dst_w.at[tid * NW], dwin_b, sem_db).wait()
            pair(j0 + W, dwin_b, 0, 1, False)
            pair(j0 + W + 2, dwin_b, 2, 3, True)

        plsc.subcore_barrier()
        pltpu.sync_copy(acc.at[pl.ds(tid * RPT, RPT)],
                        out.at[pl.ds(chunk * NP + tid * RPT, RPT)])


def _sc_propagate(hp_flat, src_c, dst_w, zeros, n_chunks):
    cps = n_chunks // 2
    mesh = plsc.VectorSubcoreMesh(core_axis_name="c", subcore_axis_name="s")
    kern = functools.partial(
        pl.kernel,
        mesh=mesh,
        out_type=jax.ShapeDtypeStruct((n_chunks * NP, 128), FN),
        scratch_types=[
            pltpu.VMEM((NB * K,), jnp.int32),
            pltpu.VMEM((W, K), jnp.int32),
            pltpu.VMEM((W, K), jnp.int32),
            pltpu.VMEM((K, 128), FN),
            pltpu.VMEM((K, 128), FN),
            pltpu.SemaphoreType.DMA,
            pltpu.SemaphoreType.DMA,
            pltpu.SemaphoreType.DMA,
            pltpu.SemaphoreType.DMA,
            pltpu.VMEM_SHARED((NP, 128), FN),
        ],
    )(functools.partial(_prop_body, cps))
    return kern(hp_flat, src_c, dst_w, zeros)


NB_HALF = NB // 2


def _deg_body(dst_t, ones_hbm, zeros, out, dst_v, ones_v, sem, acc):
    cid = lax.axis_index("c")
    tid = lax.axis_index("s")
    pltpu.sync_copy(dst_t.at[tid], dst_v)
    pltpu.sync_copy(ones_hbm, ones_v)
    pltpu.sync_copy(zeros.at[pl.ds(tid * RPT, RPT)],
                    acc.at[pl.ds(tid * RPT, RPT)])
    plsc.subcore_barrier()

    lo = cid * NB_HALF
    hi = lo + NB_HALF

    @pl.loop(lo, hi, unroll=False)
    def step(j):
        pltpu.async_copy(ones_v, acc.at[dst_v.at[j]], sem, add=True)

        @pl.when(j >= lo + 4)
        def _():
            pltpu.make_async_copy(ones_v, acc.at[dst_v.at[lo]], sem).wait()
    for _ in range(4):
        pltpu.make_async_copy(ones_v, acc.at[dst_v.at[lo]], sem).wait()
    plsc.subcore_barrier()
    pltpu.sync_copy(acc.at[pl.ds(tid * RPT, RPT)],
                    out.at[pl.ds(cid * NP + tid * RPT, RPT)])


def _sc_degree(dst_t, ones_hbm, zeros):
    mesh = plsc.VectorSubcoreMesh(core_axis_name="c", subcore_axis_name="s")
    kern = functools.partial(
        pl.kernel,
        mesh=mesh,
        out_type=jax.ShapeDtypeStruct((2 * NP, 128), FN),
        scratch_types=[
            pltpu.VMEM((NB, K), jnp.int32),
            pltpu.VMEM((K, 128), FN),
            pltpu.SemaphoreType.DMA,
            pltpu.VMEM_SHARED((NP, 128), FN),
        ],
    )(_deg_body)
    return kern(dst_t, ones_hbm, zeros)


def _rsqrt(x):
    r = lax.rsqrt(x)
    return r * (1.5 - 0.5 * x * r * r)


def _m0_body(x_ref, w_ref, out_ref):
    out_ref[...] = jnp.dot(x_ref[...], w_ref[...], preferred_element_type=FN)


def _tc_m0(xp, w0):
    return pl.pallas_call(
        _m0_body,
        grid=(GRID,),
        in_specs=[
            pl.BlockSpec((R, 256), lambda i: (i, 0)),
            pl.BlockSpec((256, 512), lambda i: (0, 0)),
        ],
        out_specs=pl.BlockSpec((R, 512), lambda i: (i, 0)),
        out_shape=jax.ShapeDtypeStruct((NP, 512), FN),
    )(xp, w0)


def _pre_body(deg_ref, m_ref, hp_ref, dis_ref):
    deg = deg_ref[0, :, 0:1] + deg_ref[1, :, 0:1]
    dis = jnp.where(deg > 0.0, _rsqrt(deg), 0.0)
    dis_ref[...] = jnp.broadcast_to(dis, (R, 128))
    for c in range(4):
        hp_ref[c] = m_ref[:, c * 128:(c + 1) * 128] * dis


def _tc_pre(deg, m0):
    return pl.pallas_call(
        _pre_body,
        grid=(GRID,),
        in_specs=[
            pl.BlockSpec((2, R, 128), lambda i: (0, i, 0)),
            pl.BlockSpec((R, 512), lambda i: (i, 0)),
        ],
        out_specs=[
            pl.BlockSpec((4, R, 128), lambda i: (0, i, 0)),
            pl.BlockSpec((R, 128), lambda i: (i, 0)),
        ],
        out_shape=[
            jax.ShapeDtypeStruct((4, NP, 128), FN),
            jax.ShapeDtypeStruct((NP, 128), FN),
        ],
    )(deg, m0)


def _mm_stats_body(c_in, prop_ref, dis_ref, b_ref, y_ref, s1_ref, s2_ref):
    i = pl.program_id(0)
    xg = jnp.concatenate([prop_ref[c] * dis_ref[...] for c in range(c_in)],
                         axis=1)
    y = xg + b_ref[...]
    y = jnp.maximum(y, 0.0)
    rows = lax.broadcasted_iota(jnp.int32, (R, 1), 0) + i * R
    y = jnp.where(rows < N, y, 0.0)
    y_ref[...] = y

    @pl.when(i == 0)
    def _():
        s1_ref[...] = jnp.zeros_like(s1_ref)
        s2_ref[...] = jnp.zeros_like(s2_ref)

    s1_ref[...] += jnp.sum(y, axis=0, keepdims=True)
    s2_ref[...] += jnp.sum(y * y, axis=0, keepdims=True)


def _tc_mm_stats(prop, dis, b, c_in, f_out):
    body = functools.partial(_mm_stats_body, c_in)
    return pl.pallas_call(
        body,
        grid=(GRID,),
        in_specs=[
            pl.BlockSpec((c_in, R, 128), lambda i: (0, i, 0)),
            pl.BlockSpec((R, 128), lambda i: (i, 0)),
            pl.BlockSpec((1, f_out), lambda i: (0, 0)),
        ],
        out_specs=[
            pl.BlockSpec((R, f_out), lambda i: (i, 0)),
            pl.BlockSpec((1, f_out), lambda i: (0, 0)),
            pl.BlockSpec((1, f_out), lambda i: (0, 0)),
        ],
        out_shape=[
            jax.ShapeDtypeStruct((NP, f_out), FN),
            jax.ShapeDtypeStruct((1, f_out), FN),
            jax.ShapeDtypeStruct((1, f_out), FN),
        ],
    )(prop, dis, b)


def _bn_next_body(relu_bn, c_out, w2, y_ref, s1_ref, s2_ref, g_ref, be_ref,
                  dis_ref, *rest):
    if w2:
        w2_ref, hp_ref = rest
    else:
        (hp_ref,) = rest
    mean = s1_ref[...] * (1.0 / N)
    var = s2_ref[...] * (1.0 / N) - mean * mean
    inv = _rsqrt(var + EPS)
    h = (y_ref[...] - mean) * inv * g_ref[...] + be_ref[...]
    if relu_bn:
        h = jnp.maximum(h, 0.0)
    if w2:
        h = jnp.dot(h, w2_ref[...], preferred_element_type=FN)
    dis = dis_ref[...]
    for c in range(c_out):
        hp_ref[c] = h[:, c * 128:(c + 1) * 128] * dis


def _tc_bn_next(y, s1, s2, g, be, dis, c_out, relu_bn, w2=None):
    f_in = y.shape[1]
    body = functools.partial(_bn_next_body, relu_bn, c_out,
                             w2 is not None)
    in_specs = [
        pl.BlockSpec((R, f_in), lambda i: (i, 0)),
        pl.BlockSpec((1, f_in), lambda i: (0, 0)),
        pl.BlockSpec((1, f_in), lambda i: (0, 0)),
        pl.BlockSpec((1, f_in), lambda i: (0, 0)),
        pl.BlockSpec((1, f_in), lambda i: (0, 0)),
        pl.BlockSpec((R, 128), lambda i: (i, 0)),
    ]
    args = [y, s1, s2, g, be, dis]
    if w2 is not None:
        in_specs.append(pl.BlockSpec(w2.shape, lambda i: (0, 0)))
        args.append(w2)
    return pl.pallas_call(
        body,
        grid=(GRID,),
        in_specs=in_specs,
        out_specs=[pl.BlockSpec((c_out, R, 128), lambda i: (0, i, 0))],
        out_shape=[jax.ShapeDtypeStruct((c_out, NP, 128), FN)],
    )(*args)[0]


def _final_body(y_ref, s1_ref, s2_ref, g_ref, be_ref, w3_ref, b3_ref,
                out_ref):
    mean = s1_ref[...] * (1.0 / N)
    var = s2_ref[...] * (1.0 / N) - mean * mean
    inv = _rsqrt(var + EPS)
    h = (y_ref[...] - mean) * inv * g_ref[...] + be_ref[...]
    h = jnp.maximum(h, 0.0)
    out = jnp.dot(h, w3_ref[...], preferred_element_type=FN) + b3_ref[...]
    out_ref[...] = jnp.maximum(out, 0.0)


def _tc_final(y, s1, s2, g, be, w3p, b3p):
    return pl.pallas_call(
        _final_body,
        grid=(GRID,),
        in_specs=[
            pl.BlockSpec((R, 256), lambda i: (i, 0)),
            pl.BlockSpec((1, 256), lambda i: (0, 0)),
            pl.BlockSpec((1, 256), lambda i: (0, 0)),
            pl.BlockSpec((1, 256), lambda i: (0, 0)),
            pl.BlockSpec((1, 256), lambda i: (0, 0)),
            pl.BlockSpec((256, 128), lambda i: (0, 0)),
            pl.BlockSpec((1, 128), lambda i: (0, 0)),
        ],
        out_specs=pl.BlockSpec((R, 128), lambda i: (i, 0)),
        out_shape=jax.ShapeDtypeStruct((NP, 128), FN),
    )(y, s1, s2, g, be, w3p, b3p)


def kernel(x, edge_index, W0, b0, g0, be0, W1, b1, g1, be1, W2, b2, g2, be2,
           W3, b3):
    si = jnp.arange(N, dtype=jnp.int32)
    src = jnp.concatenate([edge_index[0].astype(jnp.int32), si])
    dst = jnp.concatenate([edge_index[1].astype(jnp.int32), si])
    pad = EP - EF
    src = jnp.pad(src, (0, pad))
    dst = jnp.pad(dst, (0, pad), constant_values=N)
    src_t = src.reshape(NT, NB * K)
    dst_t = dst.reshape(NT, NB, K)
    dst_w = dst.reshape(NT * NW, W, K)
    src_c2 = (src_t[None] + (jnp.arange(2, dtype=jnp.int32) * NP)[:, None, None]
              ).reshape(2 * NT, NB * K)
    src_c4 = (src_t[None] + (jnp.arange(4, dtype=jnp.int32) * NP)[:, None, None]
              ).reshape(4 * NT, NB * K)

    zeros128 = jnp.zeros((NP, 128), FN)
    ones_k = jnp.ones((K, 128), FN)
    xp = jnp.pad(x, ((0, NP - N), (0, 0)))

    deg = _sc_degree(dst_t, ones_k, zeros128)
    m0 = _tc_m0(xp, W0)
    hp0, dis = _tc_pre(deg.reshape(2, NP, 128), m0)

    p0 = _sc_propagate(hp0.reshape(4 * NP, 128), src_c4, dst_w, zeros128, 4)
    y0, s01, s02 = _tc_mm_stats(p0.reshape(4, NP, 128), dis,
                                b0.reshape(1, -1), 4, 512)
    hp1 = _tc_bn_next(y0, s01, s02, g0.reshape(1, -1), be0.reshape(1, -1),
                      dis, 4, relu_bn=True, w2=W1)

    p1 = _sc_propagate(hp1.reshape(4 * NP, 128), src_c4, dst_w, zeros128, 4)
    y1, s11, s12 = _tc_mm_stats(p1.reshape(4, NP, 128), dis,
                                b1.reshape(1, -1), 4, 512)
    hp2 = _tc_bn_next(y1, s11, s12, g1.reshape(1, -1), be1.reshape(1, -1),
                      dis, 2, relu_bn=False, w2=W2)

    p2 = _sc_propagate(hp2.reshape(2 * NP, 128), src_c2, dst_w, zeros128, 2)
    y2, s21, s22 = _tc_mm_stats(p2.reshape(2, NP, 128), dis,
                                b2.reshape(1, -1), 2, 256)

    w3p = jnp.pad(W3, ((0, 0), (0, 128 - CLS)))
    b3p = jnp.pad(b3, (0, 128 - CLS)).reshape(1, 128)
    out = _tc_final(y2, s21, s22, g2.reshape(1, -1), be2.reshape(1, -1),
                    w3p, b3p)
    return out[:N, :CLS]


# speedup vs baseline: 1.3635x; 1.3635x over previous
import functools

import jax
import jax.numpy as jnp
from jax import lax
from jax.experimental import pallas as pl
from jax.experimental.pallas import tpu as pltpu
from jax.experimental.pallas import tpu_sc as plsc

FN = jnp.float32

N = 10000
NP = 10240
NT = 16
RPT = NP // NT
E_RAW = 160000
EF = E_RAW + N
K = 112
NB = 96
NB2 = NB // 2
W = 4
NW = NB // W
NW2 = NW // 2
EP = NT * NB * K
R = 1024
GRID = NP // R
EPS = 1e-3
CLS = 40


def _prop_body(cps, hp, idx_cw, zeros, out, win_a, win_b,
               rows_a, rows_b, sem_da, sem_db, sem_a, sem_b, acc):
    cid = lax.axis_index("c")
    tid = lax.axis_index("s")

    def pair(j_unused, win, ta, tb, win2, t2, last):
        pltpu.async_copy(hp.at[win.at[0, tb]], rows_b, sem_b)
        pltpu.make_async_copy(hp.at[win.at[0, ta]], rows_a, sem_a).wait()
        pltpu.sync_copy(rows_a, acc.at[win.at[1, ta]], add=True)
        if last:
            @pl.when(j_unused)
            def _():
                pltpu.async_copy(hp.at[win2.at[0, t2]], rows_a, sem_a)
        else:
            pltpu.async_copy(hp.at[win2.at[0, t2]], rows_a, sem_a)
        pltpu.make_async_copy(hp.at[win.at[0, tb]], rows_b, sem_b).wait()
        pltpu.sync_copy(rows_b, acc.at[win.at[1, tb]], add=True)

    for lc in range(cps):
        chunk = cid * cps + lc
        wbase = (chunk * NT + tid) * NW
        pltpu.sync_copy(idx_cw.at[wbase], win_a)
        pltpu.async_copy(hp.at[win_a.at[0, 0]], rows_a, sem_a)
        pltpu.sync_copy(zeros.at[pl.ds(tid * RPT, RPT)],
                        acc.at[pl.ds(tid * RPT, RPT)])
        plsc.subcore_barrier()

        @pl.loop(0, NW2, unroll=False)
        def outer(w2):
            wa = 2 * w2
            more = w2 < NW2 - 1
            pltpu.async_copy(idx_cw.at[wbase + wa + 1], win_b, sem_db)
            pair(None, win_a, 0, 1, win_a, 2, False)
            pltpu.make_async_copy(idx_cw.at[wbase], win_b, sem_db).wait()
            pair(None, win_a, 2, 3, win_b, 0, False)

            @pl.when(more)
            def _():
                pltpu.async_copy(idx_cw.at[wbase + wa + 2], win_a, sem_da)

            pair(None, win_b, 0, 1, win_b, 2, False)

            @pl.when(more)
            def _():
                pltpu.make_async_copy(idx_cw.at[wbase], win_a, sem_da).wait()

            pair(more, win_b, 2, 3, win_a, 0, True)

        plsc.subcore_barrier()
        pltpu.sync_copy(acc.at[pl.ds(tid * RPT, RPT)],
                        out.at[pl.ds(chunk * NP + tid * RPT, RPT)])


def _sc_propagate(hp_flat, idx_cw, zeros, n_chunks):
    cps = n_chunks // 2
    mesh = plsc.VectorSubcoreMesh(core_axis_name="c", subcore_axis_name="s")
    kern = functools.partial(
        pl.kernel,
        mesh=mesh,
        out_type=jax.ShapeDtypeStruct((n_chunks * NP, 128), FN),
        scratch_types=[
            pltpu.VMEM((2, W, K), jnp.int32),
            pltpu.VMEM((2, W, K), jnp.int32),
            pltpu.VMEM((K, 128), FN),
            pltpu.VMEM((K, 128), FN),
            pltpu.SemaphoreType.DMA,
            pltpu.SemaphoreType.DMA,
            pltpu.SemaphoreType.DMA,
            pltpu.SemaphoreType.DMA,
            pltpu.VMEM_SHARED((NP, 128), FN),
        ],
    )(functools.partial(_prop_body, cps))
    return kern(hp_flat, idx_cw, zeros)


NB_HALF = NB // 2


def _deg_body(dst_t, ones_hbm, zeros, out, dst_v, ones_v, sem, acc):
    cid = lax.axis_index("c")
    tid = lax.axis_index("s")
    pltpu.sync_copy(dst_t.at[tid], dst_v)
    pltpu.sync_copy(ones_hbm, ones_v)
    pltpu.sync_copy(zeros.at[pl.ds(tid * RPT, RPT)],
                    acc.at[pl.ds(tid * RPT, RPT)])
    plsc.subcore_barrier()

    lo = cid * NB_HALF
    hi = lo + NB_HALF

    @pl.loop(lo, hi, unroll=False)
    def step(j):
        pltpu.async_copy(ones_v, acc.at[dst_v.at[j]], sem, add=True)

        @pl.when(j >= lo + 4)
        def _():
            pltpu.make_async_copy(ones_v, acc.at[dst_v.at[lo]], sem).wait()
    for _ in range(4):
        pltpu.make_async_copy(ones_v, acc.at[dst_v.at[lo]], sem).wait()
    plsc.subcore_barrier()
    pltpu.sync_copy(acc.at[pl.ds(tid * RPT, RPT)],
                    out.at[pl.ds(cid * NP + tid * RPT, RPT)])


def _sc_degree(dst_t, ones_hbm, zeros):
    mesh = plsc.VectorSubcoreMesh(core_axis_name="c", subcore_axis_name="s")
    kern = functools.partial(
        pl.kernel,
        mesh=mesh,
        out_type=jax.ShapeDtypeStruct((2 * NP, 128), FN),
        scratch_types=[
            pltpu.VMEM((NB, K), jnp.int32),
            pltpu.VMEM((K, 128), FN),
            pltpu.SemaphoreType.DMA,
            pltpu.VMEM_SHARED((NP, 128), FN),
        ],
    )(_deg_body)
    return kern(dst_t, ones_hbm, zeros)


def _rsqrt(x):
    r = lax.rsqrt(x)
    return r * (1.5 - 0.5 * x * r * r)


def _m0_body(x_ref, w_ref, out_ref):
    out_ref[...] = jnp.dot(x_ref[...], w_ref[...], preferred_element_type=FN)


def _tc_m0(xp, w0):
    return pl.pallas_call(
        _m0_body,
        grid=(GRID,),
        in_specs=[
            pl.BlockSpec((R, 256), lambda i: (i, 0)),
            pl.BlockSpec((256, 512), lambda i: (0, 0)),
        ],
        out_specs=pl.BlockSpec((R, 512), lambda i: (i, 0)),
        out_shape=jax.ShapeDtypeStruct((NP, 512), FN),
    )(xp, w0)


def _pre_body(deg_ref, m_ref, hp_ref, dis_ref):
    deg = deg_ref[0, :, 0:1] + deg_ref[1, :, 0:1]
    dis = jnp.where(deg > 0.0, _rsqrt(deg), 0.0)
    dis_ref[...] = jnp.broadcast_to(dis, (R, 128))
    for c in range(4):
        hp_ref[c] = m_ref[:, c * 128:(c + 1) * 128] * dis


def _tc_pre(deg, m0):
    return pl.pallas_call(
        _pre_body,
        grid=(GRID,),
        in_specs=[
            pl.BlockSpec((2, R, 128), lambda i: (0, i, 0)),
            pl.BlockSpec((R, 512), lambda i: (i, 0)),
        ],
        out_specs=[
            pl.BlockSpec((4, R, 128), lambda i: (0, i, 0)),
            pl.BlockSpec((R, 128), lambda i: (i, 0)),
        ],
        out_shape=[
            jax.ShapeDtypeStruct((4, NP, 128), FN),
            jax.ShapeDtypeStruct((NP, 128), FN),
        ],
    )(deg, m0)


def _mm_stats_body(c_in, prop_ref, dis_ref, b_ref, y_ref, s1_ref, s2_ref):
    i = pl.program_id(0)
    xg = jnp.concatenate([prop_ref[c] * dis_ref[...] for c in range(c_in)],
                         axis=1)
    y = xg + b_ref[...]
    y = jnp.maximum(y, 0.0)
    rows = lax.broadcasted_iota(jnp.int32, (R, 1), 0) + i * R
    y = jnp.where(rows < N, y, 0.0)
    y_ref[...] = y

    @pl.when(i == 0)
    def _():
        s1_ref[...] = jnp.zeros_like(s1_ref)
        s2_ref[...] = jnp.zeros_like(s2_ref)

    s1_ref[...] += jnp.sum(y, axis=0, keepdims=True)
    s2_ref[...] += jnp.sum(y * y, axis=0, keepdims=True)


def _tc_mm_stats(prop, dis, b, c_in, f_out):
    body = functools.partial(_mm_stats_body, c_in)
    return pl.pallas_call(
        body,
        grid=(GRID,),
        in_specs=[
            pl.BlockSpec((c_in, R, 128), lambda i: (0, i, 0)),
            pl.BlockSpec((R, 128), lambda i: (i, 0)),
            pl.BlockSpec((1, f_out), lambda i: (0, 0)),
        ],
        out_specs=[
            pl.BlockSpec((R, f_out), lambda i: (i, 0)),
            pl.BlockSpec((1, f_out), lambda i: (0, 0)),
            pl.BlockSpec((1, f_out), lambda i: (0, 0)),
        ],
        out_shape=[
            jax.ShapeDtypeStruct((NP, f_out), FN),
            jax.ShapeDtypeStruct((1, f_out), FN),
            jax.ShapeDtypeStruct((1, f_out), FN),
        ],
    )(prop, dis, b)


def _bn_next_body(relu_bn, c_out, w2, y_ref, s1_ref, s2_ref, g_ref, be_ref,
                  dis_ref, *rest):
    if w2:
        w2_ref, hp_ref = rest
    else:
        (hp_ref,) = rest
    mean = s1_ref[...] * (1.0 / N)
    var = s2_ref[...] * (1.0 / N) - mean * mean
    inv = _rsqrt(var + EPS)
    h = (y_ref[...] - mean) * inv * g_ref[...] + be_ref[...]
    if relu_bn:
        h = jnp.maximum(h, 0.0)
    if w2:
        h = jnp.dot(h, w2_ref[...], preferred_element_type=FN)
    dis = dis_ref[...]
    for c in range(c_out):
        hp_ref[c] = h[:, c * 128:(c + 1) * 128] * dis


def _tc_bn_next(y, s1, s2, g, be, dis, c_out, relu_bn, w2=None):
    f_in = y.shape[1]
    body = functools.partial(_bn_next_body, relu_bn, c_out,
                             w2 is not None)
    in_specs = [
        pl.BlockSpec((R, f_in), lambda i: (i, 0)),
        pl.BlockSpec((1, f_in), lambda i: (0, 0)),
        pl.BlockSpec((1, f_in), lambda i: (0, 0)),
        pl.BlockSpec((1, f_in), lambda i: (0, 0)),
        pl.BlockSpec((1, f_in), lambda i: (0, 0)),
        pl.BlockSpec((R, 128), lambda i: (i, 0)),
    ]
    args = [y, s1, s2, g, be, dis]
    if w2 is not None:
        in_specs.append(pl.BlockSpec(w2.shape, lambda i: (0, 0)))
        args.append(w2)
    return pl.pallas_call(
        body,
        grid=(GRID,),
        in_specs=in_specs,
        out_specs=[pl.BlockSpec((c_out, R, 128), lambda i: (0, i, 0))],
        out_shape=[jax.ShapeDtypeStruct((c_out, NP, 128), FN)],
    )(*args)[0]


def _final_body(y_ref, s1_ref, s2_ref, g_ref, be_ref, w3_ref, b3_ref,
                out_ref):
    mean = s1_ref[...] * (1.0 / N)
    var = s2_ref[...] * (1.0 / N) - mean * mean
    inv = _rsqrt(var + EPS)
    h = (y_ref[...] - mean) * inv * g_ref[...] + be_ref[...]
    h = jnp.maximum(h, 0.0)
    out = jnp.dot(h, w3_ref[...], preferred_element_type=FN) + b3_ref[...]
    out_ref[...] = jnp.maximum(out, 0.0)


def _tc_final(y, s1, s2, g, be, w3p, b3p):
    return pl.pallas_call(
        _final_body,
        grid=(GRID,),
        in_specs=[
            pl.BlockSpec((R, 256), lambda i: (i, 0)),
            pl.BlockSpec((1, 256), lambda i: (0, 0)),
            pl.BlockSpec((1, 256), lambda i: (0, 0)),
            pl.BlockSpec((1, 256), lambda i: (0, 0)),
            pl.BlockSpec((1, 256), lambda i: (0, 0)),
            pl.BlockSpec((256, 128), lambda i: (0, 0)),
            pl.BlockSpec((1, 128), lambda i: (0, 0)),
        ],
        out_specs=pl.BlockSpec((R, 128), lambda i: (i, 0)),
        out_shape=jax.ShapeDtypeStruct((NP, 128), FN),
    )(y, s1, s2, g, be, w3p, b3p)


def kernel(x, edge_index, W0, b0, g0, be0, W1, b1, g1, be1, W2, b2, g2, be2,
           W3, b3):
    si = jnp.arange(N, dtype=jnp.int32)
    src = jnp.concatenate([edge_index[0].astype(jnp.int32), si])
    dst = jnp.concatenate([edge_index[1].astype(jnp.int32), si])
    pad = EP - EF
    src = jnp.pad(src, (0, pad))
    dst = jnp.pad(dst, (0, pad), constant_values=N)
    dst_t = dst.reshape(NT, NB, K)
    src_w = src.reshape(NT, NW, W, K)
    dst_w = dst.reshape(NT, NW, W, K)

    def make_idx(c):
        s = (src_w[None] +
             (jnp.arange(c, dtype=jnp.int32) * NP)[:, None, None, None, None])
        d = jnp.broadcast_to(dst_w[None], (c,) + dst_w.shape)
        return jnp.stack([s, d], axis=3).reshape(c * NT * NW, 2, W, K)

    idx2 = make_idx(2)
    idx4 = make_idx(4)

    zeros128 = jnp.zeros((NP, 128), FN)
    ones_k = jnp.ones((K, 128), FN)
    xp = jnp.pad(x, ((0, NP - N), (0, 0)))

    deg = _sc_degree(dst_t, ones_k, zeros128)
    m0 = _tc_m0(xp, W0)
    hp0, dis = _tc_pre(deg.reshape(2, NP, 128), m0)

    p0 = _sc_propagate(hp0.reshape(4 * NP, 128), idx4, zeros128, 4)
    y0, s01, s02 = _tc_mm_stats(p0.reshape(4, NP, 128), dis,
                                b0.reshape(1, -1), 4, 512)
    hp1 = _tc_bn_next(y0, s01, s02, g0.reshape(1, -1), be0.reshape(1, -1),
                      dis, 4, relu_bn=True, w2=W1)

    p1 = _sc_propagate(hp1.reshape(4 * NP, 128), idx4, zeros128, 4)
    y1, s11, s12 = _tc_mm_stats(p1.reshape(4, NP, 128), dis,
                                b1.reshape(1, -1), 4, 512)
    hp2 = _tc_bn_next(y1, s11, s12, g1.reshape(1, -1), be1.reshape(1, -1),
                      dis, 2, relu_bn=False, w2=W2)

    p2 = _sc_propagate(hp2.reshape(2 * NP, 128), idx2, zeros128, 2)
    y2, s21, s22 = _tc_mm_stats(p2.reshape(2, NP, 128), dis,
                                b2.reshape(1, -1), 2, 256)

    w3p = jnp.pad(W3, ((0, 0), (0, 128 - CLS)))
    b3p = jnp.pad(b3, (0, 128 - CLS)).reshape(1, 128)
    out = _tc_final(y2, s21, s22, g2.reshape(1, -1), be2.reshape(1, -1),
                    w3p, b3p)
    return out[:N, :CLS]
